# Initial kernel scaffold; baseline (speedup 1.0000x reference)
import jax
import jax.numpy as jnp
from jax.experimental import pallas as pl


def kernel(proposed_roi_bboxes, predicted_roi_loc, predicted_roi_score):
    raise NotImplementedError("write your pallas kernel here")


# speedup vs baseline: 28.6303x; 28.6303x over previous
import jax
import jax.numpy as jnp
from jax import lax
from jax.experimental import pallas as pl

_N = 5000
_NPAD = 5120
_C = 20
_K = 200
_KPAD = 256
_IMG_H = 600.0
_IMG_W = 800.0
_SCORE_THR = 0.05
_NMS_THR = 0.3


def _body(score_ref, prop_ref, dy_ref, dx_ref, dh_ref, dw_ref,
          by1_ref, bx1_ref, by2_ref, bx2_ref, sc_ref, lb_ref):
    score = score_ref[...]
    m = jnp.max(score, axis=0, keepdims=True)
    e = jnp.exp(score - m)
    prob = e / jnp.sum(e, axis=0, keepdims=True)

    lane = lax.broadcasted_iota(jnp.int32, (_C, _NPAD), 1)
    valid = lane < _N
    probc = prob[1:, :]
    masked = jnp.where((probc > _SCORE_THR) & valid, probc, -jnp.inf)

    prop = prop_ref[...]
    h = prop[2:3] - prop[0:1]
    w = prop[3:4] - prop[1:2]
    cy = prop[0:1] + 0.5 * h
    cx = prop[1:2] + 0.5 * w

    dy = dy_ref[...] * 0.1
    dx = dx_ref[...] * 0.1
    dh = dh_ref[...] * 0.2
    dw = dw_ref[...] * 0.2
    cy2 = dy * h + cy
    cx2 = dx * w + cx
    h2 = jnp.exp(dh) * h
    w2 = jnp.exp(dw) * w
    y1 = jnp.clip(cy2 - 0.5 * h2, 0.0, _IMG_H)
    x1 = jnp.clip(cx2 - 0.5 * w2, 0.0, _IMG_W)
    y2 = jnp.clip(cy2 + 0.5 * h2, 0.0, _IMG_H)
    x2 = jnp.clip(cx2 + 0.5 * w2, 0.0, _IMG_W)

    col = lax.broadcasted_iota(jnp.int32, (_C, _KPAD), 1)
    zk = jnp.zeros((_C, _KPAD), jnp.float32)

    def sel_body(t, carry):
        msk, vals, gy1, gx1, gy2, gx2 = carry
        mx = jnp.max(msk, axis=1, keepdims=True)
        cand = jnp.where(msk == mx, lane, _NPAD)
        sel = jnp.min(cand, axis=1, keepdims=True)
        onehot = lane == sel
        ohf = onehot.astype(jnp.float32)
        sy1 = jnp.sum(ohf * y1, axis=1, keepdims=True)
        sx1 = jnp.sum(ohf * x1, axis=1, keepdims=True)
        sy2 = jnp.sum(ohf * y2, axis=1, keepdims=True)
        sx2 = jnp.sum(ohf * x2, axis=1, keepdims=True)
        att = col == t
        vals = jnp.where(att, mx, vals)
        gy1 = jnp.where(att, sy1, gy1)
        gx1 = jnp.where(att, sx1, gx1)
        gy2 = jnp.where(att, sy2, gy2)
        gx2 = jnp.where(att, sx2, gx2)
        msk = jnp.where(onehot, -jnp.inf, msk)
        return msk, vals, gy1, gx1, gy2, gx2

    init = (masked, jnp.full((_C, _KPAD), -jnp.inf, jnp.float32),
            zk, zk, zk, zk)
    _, vals, gy1, gx1, gy2, gx2 = lax.fori_loop(0, _K, sel_body, init)

    area = jnp.maximum(gy2 - gy1, 0.0) * jnp.maximum(gx2 - gx1, 0.0)

    def nms_body(i, keep):
        ohf = (col == i).astype(jnp.float32)
        by1 = jnp.sum(ohf * gy1, axis=1, keepdims=True)
        bx1 = jnp.sum(ohf * gx1, axis=1, keepdims=True)
        by2 = jnp.sum(ohf * gy2, axis=1, keepdims=True)
        bx2 = jnp.sum(ohf * gx2, axis=1, keepdims=True)
        bkeep = jnp.sum(ohf * keep, axis=1, keepdims=True)
        area_i = jnp.maximum(by2 - by1, 0.0) * jnp.maximum(bx2 - bx1, 0.0)
        yy1 = jnp.maximum(by1, gy1)
        xx1 = jnp.maximum(bx1, gx1)
        yy2 = jnp.minimum(by2, gy2)
        xx2 = jnp.minimum(bx2, gx2)
        inter = jnp.maximum(yy2 - yy1, 0.0) * jnp.maximum(xx2 - xx1, 0.0)
        iou = inter / (area_i + area - inter + 1e-9)
        supf = jnp.where((iou > _NMS_THR) & (col > i), bkeep, 0.0)
        return keep * (1.0 - supf)

    keep = lax.fori_loop(0, _K, nms_body, jnp.ones((_C, _KPAD), jnp.float32))
    final = (keep > 0.5) & (vals > _SCORE_THR)

    by1_ref[...] = jnp.where(final, gy1, 0.0)
    bx1_ref[...] = jnp.where(final, gx1, 0.0)
    by2_ref[...] = jnp.where(final, gy2, 0.0)
    bx2_ref[...] = jnp.where(final, gx2, 0.0)
    sc_ref[...] = jnp.where(final, vals, 0.0)
    crow = lax.broadcasted_iota(jnp.int32, (_C, _KPAD), 0)
    lb_ref[...] = jnp.where(final, crow + 1, 0)


def kernel(proposed_roi_bboxes, predicted_roi_loc, predicted_roi_score):
    pad = _NPAD - _N
    scoreT = jnp.pad(predicted_roi_score.T, ((0, 0), (0, pad)))
    propT = jnp.pad(proposed_roi_bboxes.T, ((0, 0), (0, pad)))
    lr = predicted_roi_loc.reshape(_N, _C + 1, 4)[:, 1:, :]
    dyT = jnp.pad(lr[..., 0].T, ((0, 0), (0, pad)))
    dxT = jnp.pad(lr[..., 1].T, ((0, 0), (0, pad)))
    dhT = jnp.pad(lr[..., 2].T, ((0, 0), (0, pad)))
    dwT = jnp.pad(lr[..., 3].T, ((0, 0), (0, pad)))

    f = jax.ShapeDtypeStruct((_C, _KPAD), jnp.float32)
    i = jax.ShapeDtypeStruct((_C, _KPAD), jnp.int32)
    by1, bx1, by2, bx2, sc, lb = pl.pallas_call(
        _body,
        out_shape=(f, f, f, f, f, i),
    )(scoreT, propT, dyT, dxT, dhT, dwT)

    bboxes = jnp.stack([by1, bx1, by2, bx2], axis=-1)[:, :_K, :]
    bboxes = bboxes.reshape(_C * _K, 4)
    labels = lb[:, :_K].reshape(_C * _K)
    scores = sc[:, :_K].reshape(_C * _K)
    return bboxes, labels, scores


# speedup vs baseline: 40.0911x; 1.4003x over previous
import functools
import jax
import jax.numpy as jnp
from jax import lax
from jax.experimental import pallas as pl
from jax.experimental.pallas import tpu as pltpu
from jax.experimental.pallas import tpu_sc as plsc

_N = 5000
_NPAD = 5120
_C = 20
_K = 200
_KPAD = 256
_IMG_H = 600.0
_IMG_W = 800.0
_SCORE_THR = 0.05
_NMS_THR = 0.3
_NEG = -1e30


def _dense_body(score_ref, prop_ref, dy_ref, dx_ref, dh_ref, dw_ref,
                m_ref, y1_ref, x1_ref, y2_ref, x2_ref):
    score = score_ref[...]
    mx = jnp.max(score, axis=0, keepdims=True)
    e = jnp.exp(score - mx)
    prob = e / jnp.sum(e, axis=0, keepdims=True)

    lane = lax.broadcasted_iota(jnp.int32, (_C, _NPAD), 1)
    valid = lane < _N
    probc = prob[1:, :]
    m_ref[...] = jnp.where((probc > _SCORE_THR) & valid, probc, -1.0)

    prop = prop_ref[...]
    h = prop[2:3] - prop[0:1]
    w = prop[3:4] - prop[1:2]
    cy = prop[0:1] + 0.5 * h
    cx = prop[1:2] + 0.5 * w

    dy = dy_ref[...] * 0.1
    dx = dx_ref[...] * 0.1
    dh = dh_ref[...] * 0.2
    dw = dw_ref[...] * 0.2
    cy2 = dy * h + cy
    cx2 = dx * w + cx
    h2 = jnp.exp(dh) * h
    w2 = jnp.exp(dw) * w
    y1_ref[...] = jnp.clip(cy2 - 0.5 * h2, 0.0, _IMG_H)
    x1_ref[...] = jnp.clip(cx2 - 0.5 * w2, 0.0, _IMG_W)
    y2_ref[...] = jnp.clip(cy2 + 0.5 * h2, 0.0, _IMG_H)
    x2_ref[...] = jnp.clip(cx2 + 0.5 * w2, 0.0, _IMG_W)


def _sc_body(m_hbm, y1_hbm, x1_hbm, y2_hbm, x2_hbm,
             ov_hbm, oy1_hbm, ox1_hbm, oy2_hbm, ox2_hbm,
             m_v, y1_v, x1_v, y2_v, x2_v, cval_v, cidx_v,
             gval_v, oidx_v, gy1_v, gx1_v, gy2_v, gx2_v,
             area_v, keep_v, ov_v, oy1_v, ox1_v, oy2_v, ox2_v):
    wid = lax.axis_index("s") * 2 + lax.axis_index("c")
    iota16 = lax.iota(jnp.int32, 16)

    @pl.when(wid < _C)
    def _():
        c = wid
        pltpu.sync_copy(m_hbm.at[c], m_v)
        pltpu.sync_copy(y1_hbm.at[c], y1_v)
        pltpu.sync_copy(x1_hbm.at[c], x1_v)
        pltpu.sync_copy(y2_hbm.at[c], y2_v)
        pltpu.sync_copy(x2_hbm.at[c], x2_v)

        def comp_body(j, n):
            idx = j * 16 + iota16
            v = plsc.load_gather(m_v, [idx])
            msk = v > _SCORE_THR
            ranks = n + plsc.cumsum(msk.astype(jnp.int32)) - 1
            plsc.store_scatter(cval_v, [ranks], v, mask=msk)
            plsc.store_scatter(cidx_v, [ranks], idx, mask=msk)
            cnt = plsc.all_reduce_population_count(msk)
            return n + jnp.max(cnt)

        n = lax.fori_loop(0, _NPAD // 16, comp_body, jnp.int32(0))
        jmax = (n + 15) // 16

        for j in range(_KPAD // 16):
            sl = pl.ds(j * 16, 16)
            gval_v[sl] = jnp.zeros((16,), jnp.float32)
            oidx_v[sl] = jnp.zeros((16,), jnp.int32)
            keep_v[sl] = jnp.ones((16,), jnp.float32)

        def sel_t(t, x):
            def scan_j(j, best):
                bv, bj = best
                v = plsc.load_gather(cval_v, [j * 16 + iota16])
                m = jnp.max(v)
                better = m > bv
                return (jnp.where(better, m, bv), jnp.where(better, j, bj))

            bv, bj = lax.fori_loop(0, jmax, scan_j,
                                   (jnp.float32(_NEG), jnp.int32(0)))

            @pl.when(bv > _SCORE_THR)
            def _():
                v = plsc.load_gather(cval_v, [bj * 16 + iota16])
                lane = jnp.max(plsc.all_reduce_ffs(v == bv))
                pos = bj * 16 + lane
                posv = jnp.full((16,), pos, jnp.int32)
                orig = jnp.max(plsc.load_gather(cidx_v, [posv]))
                lane0 = iota16 == 0
                tv = jnp.full((16,), t, jnp.int32)
                plsc.store_scatter(gval_v, [tv],
                                   jnp.full((16,), bv, jnp.float32),
                                   mask=lane0)
                plsc.store_scatter(oidx_v, [tv],
                                   jnp.full((16,), orig, jnp.int32),
                                   mask=lane0)
                plsc.store_scatter(cval_v, [posv],
                                   jnp.full((16,), _NEG, jnp.float32),
                                   mask=lane0)
            return x

        lax.fori_loop(0, _K, sel_t, jnp.int32(0))

        for j in range(13):
            sl = pl.ds(j * 16, 16)
            oi = oidx_v[sl]
            vy1 = plsc.load_gather(y1_v, [oi])
            vx1 = plsc.load_gather(x1_v, [oi])
            vy2 = plsc.load_gather(y2_v, [oi])
            vx2 = plsc.load_gather(x2_v, [oi])
            gy1_v[sl] = vy1
            gx1_v[sl] = vx1
            gy2_v[sl] = vy2
            gx2_v[sl] = vx2
            area_v[sl] = (jnp.maximum(vy2 - vy1, 0.0)
                          * jnp.maximum(vx2 - vx1, 0.0))

        def nms_i(i, x):
            iv = jnp.full((16,), i, jnp.int32)
            by1 = jnp.max(plsc.load_gather(gy1_v, [iv]))
            bx1 = jnp.max(plsc.load_gather(gx1_v, [iv]))
            by2 = jnp.max(plsc.load_gather(gy2_v, [iv]))
            bx2 = jnp.max(plsc.load_gather(gx2_v, [iv]))
            ai = jnp.max(plsc.load_gather(area_v, [iv]))
            bk = jnp.max(plsc.load_gather(keep_v, [iv]))

            @pl.when(bk > 0.5)
            def _():
                for j in range(13):
                    sl = pl.ds(j * 16, 16)
                    yy1 = jnp.maximum(by1, gy1_v[sl])
                    xx1 = jnp.maximum(bx1, gx1_v[sl])
                    yy2 = jnp.minimum(by2, gy2_v[sl])
                    xx2 = jnp.minimum(bx2, gx2_v[sl])
                    inter = (jnp.maximum(yy2 - yy1, 0.0)
                             * jnp.maximum(xx2 - xx1, 0.0))
                    iou = inter / (ai + area_v[sl] - inter + 1e-9)
                    colv = j * 16 + iota16
                    sup = (iou > _NMS_THR) & (colv > i)
                    keep_v[sl] = jnp.where(sup, 0.0, keep_v[sl])
            return x

        lax.fori_loop(0, _K, nms_i, jnp.int32(0))

        for j in range(_KPAD // 16):
            sl = pl.ds(j * 16, 16)
            if j < 13:
                fin = (keep_v[sl] > 0.5) & (gval_v[sl] > _SCORE_THR)
                ov_v[sl] = jnp.where(fin, gval_v[sl], 0.0)
                oy1_v[sl] = jnp.where(fin, gy1_v[sl], 0.0)
                ox1_v[sl] = jnp.where(fin, gx1_v[sl], 0.0)
                oy2_v[sl] = jnp.where(fin, gy2_v[sl], 0.0)
                ox2_v[sl] = jnp.where(fin, gx2_v[sl], 0.0)
            else:
                z = jnp.zeros((16,), jnp.float32)
                ov_v[sl] = z
                oy1_v[sl] = z
                ox1_v[sl] = z
                oy2_v[sl] = z
                ox2_v[sl] = z
        pltpu.sync_copy(ov_v, ov_hbm.at[c])
        pltpu.sync_copy(oy1_v, oy1_hbm.at[c])
        pltpu.sync_copy(ox1_v, ox1_hbm.at[c])
        pltpu.sync_copy(oy2_v, oy2_hbm.at[c])
        pltpu.sync_copy(ox2_v, ox2_hbm.at[c])


def _sc_stage(m, y1, x1, y2, x2):
    fo = jax.ShapeDtypeStruct((_C, _KPAD), jnp.float32)
    fN = lambda: pltpu.VMEM((_NPAD,), jnp.float32)
    fK = lambda: pltpu.VMEM((_KPAD,), jnp.float32)
    kern = pl.kernel(
        _sc_body,
        out_type=(fo, fo, fo, fo, fo),
        mesh=plsc.VectorSubcoreMesh(core_axis_name="c", subcore_axis_name="s",
                                    num_cores=2, num_subcores=16),
        compiler_params=pltpu.CompilerParams(use_tc_tiling_on_sc=False,
                                             needs_layout_passes=False),
        scratch_types=[
            fN(), fN(), fN(), fN(), fN(),
            pltpu.VMEM((_NPAD,), jnp.float32),
            pltpu.VMEM((_NPAD,), jnp.int32),
            fK(), pltpu.VMEM((_KPAD,), jnp.int32),
            fK(), fK(), fK(), fK(),
            fK(), fK(),
            fK(), fK(), fK(), fK(), fK(),
        ],
    )
    return kern(m, y1, x1, y2, x2)


def kernel(proposed_roi_bboxes, predicted_roi_loc, predicted_roi_score):
    pad = _NPAD - _N
    scoreT = jnp.pad(predicted_roi_score.T, ((0, 0), (0, pad)))
    propT = jnp.pad(proposed_roi_bboxes.T, ((0, 0), (0, pad)))
    lr = predicted_roi_loc.reshape(_N, _C + 1, 4)[:, 1:, :]
    dyT = jnp.pad(lr[..., 0].T, ((0, 0), (0, pad)))
    dxT = jnp.pad(lr[..., 1].T, ((0, 0), (0, pad)))
    dhT = jnp.pad(lr[..., 2].T, ((0, 0), (0, pad)))
    dwT = jnp.pad(lr[..., 3].T, ((0, 0), (0, pad)))

    g = jax.ShapeDtypeStruct((_C, _NPAD), jnp.float32)
    m, y1, x1, y2, x2 = pl.pallas_call(
        _dense_body,
        out_shape=(g, g, g, g, g),
    )(scoreT, propT, dyT, dxT, dhT, dwT)

    ov, oy1, ox1, oy2, ox2 = _sc_stage(m, y1, x1, y2, x2)

    bboxes = jnp.stack([oy1, ox1, oy2, ox2], axis=-1)[:, :_K, :]
    bboxes = bboxes.reshape(_C * _K, 4)
    scores = ov[:, :_K].reshape(_C * _K)
    crow = jnp.arange(1, _C + 1, dtype=jnp.int32)[:, None]
    labels = jnp.where(ov[:, :_K] > 0.0, crow, 0).reshape(_C * _K)
    return bboxes, labels, scores


# speedup vs baseline: 53.4292x; 1.3327x over previous
import functools
import jax
import jax.numpy as jnp
from jax import lax
from jax.experimental import pallas as pl
from jax.experimental.pallas import tpu as pltpu
from jax.experimental.pallas import tpu_sc as plsc

_N = 5000
_NPAD = 5120
_C = 20
_K = 200
_KPAD = 256
_IMG_H = 600.0
_IMG_W = 800.0
_SCORE_THR = 0.05
_NMS_THR = 0.3
_NEG = -1e30


def _dense_body(score_ref, prop_ref, dy_ref, dx_ref, dh_ref, dw_ref,
                m_ref, y1_ref, x1_ref, y2_ref, x2_ref):
    score = score_ref[...]
    mx = jnp.max(score, axis=0, keepdims=True)
    e = jnp.exp(score - mx)
    prob = e / jnp.sum(e, axis=0, keepdims=True)

    lane = lax.broadcasted_iota(jnp.int32, (_C, _NPAD), 1)
    valid = lane < _N
    probc = prob[1:, :]
    m_ref[...] = jnp.where((probc > _SCORE_THR) & valid, probc, -1.0)

    prop = prop_ref[...]
    h = prop[2:3] - prop[0:1]
    w = prop[3:4] - prop[1:2]
    cy = prop[0:1] + 0.5 * h
    cx = prop[1:2] + 0.5 * w

    dy = dy_ref[...] * 0.1
    dx = dx_ref[...] * 0.1
    dh = dh_ref[...] * 0.2
    dw = dw_ref[...] * 0.2
    cy2 = dy * h + cy
    cx2 = dx * w + cx
    h2 = jnp.exp(dh) * h
    w2 = jnp.exp(dw) * w
    y1_ref[...] = jnp.clip(cy2 - 0.5 * h2, 0.0, _IMG_H)
    x1_ref[...] = jnp.clip(cx2 - 0.5 * w2, 0.0, _IMG_W)
    y2_ref[...] = jnp.clip(cy2 + 0.5 * h2, 0.0, _IMG_H)
    x2_ref[...] = jnp.clip(cx2 + 0.5 * w2, 0.0, _IMG_W)


def _sc_body(m_hbm, y1_hbm, x1_hbm, y2_hbm, x2_hbm,
             ov_hbm, oy1_hbm, ox1_hbm, oy2_hbm, ox2_hbm,
             m_v, y1_v, x1_v, y2_v, x2_v, cval_v, cidx_v, vmax_v,
             gval_v, oidx_v, gy1_v, gx1_v, gy2_v, gx2_v,
             area_v, keep_v, ov_v, oy1_v, ox1_v, oy2_v, ox2_v):
    wid = lax.axis_index("s") * 2 + lax.axis_index("c")
    iota16 = lax.iota(jnp.int32, 16)

    @pl.when(wid < _C)
    def _():
        c = wid
        pltpu.sync_copy(m_hbm.at[c], m_v)
        pltpu.sync_copy(y1_hbm.at[c], y1_v)
        pltpu.sync_copy(x1_hbm.at[c], x1_v)
        pltpu.sync_copy(y2_hbm.at[c], y2_v)
        pltpu.sync_copy(x2_hbm.at[c], x2_v)

        def comp_body(j, n):
            idx = j * 16 + iota16
            v = plsc.load_gather(m_v, [idx])
            msk = v > _SCORE_THR
            ranks = n + plsc.cumsum(msk.astype(jnp.int32)) - 1
            plsc.store_scatter(cval_v, [ranks], v, mask=msk)
            plsc.store_scatter(cidx_v, [ranks], idx, mask=msk)
            cnt = plsc.all_reduce_population_count(msk)
            return n + jnp.max(cnt)

        n = lax.fori_loop(0, _NPAD // 16, comp_body, jnp.int32(0))
        jmax = (n + 15) // 16
        qmax = (jmax + 15) // 16
        lane0 = iota16 == 0

        tail = n + iota16
        plsc.store_scatter(cval_v, [tail],
                           jnp.full((16,), _NEG, jnp.float32),
                           mask=tail < _NPAD)

        for j in range(_KPAD // 16):
            sl = pl.ds(j * 16, 16)
            gval_v[sl] = jnp.zeros((16,), jnp.float32)
            oidx_v[sl] = jnp.zeros((16,), jnp.int32)
            keep_v[sl] = jnp.ones((16,), jnp.float32)

        for q in range(_NPAD // 256):
            vmax_v[pl.ds(q * 16, 16)] = jnp.full((16,), _NEG, jnp.float32)

        def vm_body(j, x):
            v = plsc.load_gather(cval_v, [j * 16 + iota16])
            plsc.store_scatter(vmax_v, [jnp.full((16,), j, jnp.int32)],
                               jnp.full((16,), jnp.max(v), jnp.float32),
                               mask=lane0)
            return x

        lax.fori_loop(0, jmax, vm_body, jnp.int32(0))

        def sel_t(t, x):
            def scan_q(q, best):
                bv, bq = best
                w = plsc.load_gather(vmax_v, [q * 16 + iota16])
                m = jnp.max(w)
                better = m > bv
                return (jnp.where(better, m, bv), jnp.where(better, q, bq))

            bv, bq = lax.fori_loop(0, qmax, scan_q,
                                   (jnp.float32(_NEG), jnp.int32(0)))

            @pl.when(bv > _SCORE_THR)
            def _():
                w = plsc.load_gather(vmax_v, [bq * 16 + iota16])
                bj = bq * 16 + jnp.max(plsc.all_reduce_ffs(w == bv))
                v = plsc.load_gather(cval_v, [bj * 16 + iota16])
                lane = jnp.max(plsc.all_reduce_ffs(v == bv))
                pos = bj * 16 + lane
                posv = jnp.full((16,), pos, jnp.int32)
                orig = jnp.max(plsc.load_gather(cidx_v, [posv]))
                tv = jnp.full((16,), t, jnp.int32)
                plsc.store_scatter(gval_v, [tv],
                                   jnp.full((16,), bv, jnp.float32),
                                   mask=lane0)
                plsc.store_scatter(oidx_v, [tv],
                                   jnp.full((16,), orig, jnp.int32),
                                   mask=lane0)
                v2 = jnp.where(iota16 == lane, _NEG, v)
                plsc.store_scatter(cval_v, [bj * 16 + iota16], v2)
                plsc.store_scatter(vmax_v, [jnp.full((16,), bj, jnp.int32)],
                                   jnp.full((16,), jnp.max(v2), jnp.float32),
                                   mask=lane0)
            return x

        lax.fori_loop(0, _K, sel_t, jnp.int32(0))

        for j in range(13):
            sl = pl.ds(j * 16, 16)
            oi = oidx_v[sl]
            vy1 = plsc.load_gather(y1_v, [oi])
            vx1 = plsc.load_gather(x1_v, [oi])
            vy2 = plsc.load_gather(y2_v, [oi])
            vx2 = plsc.load_gather(x2_v, [oi])
            gy1_v[sl] = vy1
            gx1_v[sl] = vx1
            gy2_v[sl] = vy2
            gx2_v[sl] = vx2
            area_v[sl] = (jnp.maximum(vy2 - vy1, 0.0)
                          * jnp.maximum(vx2 - vx1, 0.0))

        def nms_i(i, x):
            iv = jnp.full((16,), i, jnp.int32)
            by1 = jnp.max(plsc.load_gather(gy1_v, [iv]))
            bx1 = jnp.max(plsc.load_gather(gx1_v, [iv]))
            by2 = jnp.max(plsc.load_gather(gy2_v, [iv]))
            bx2 = jnp.max(plsc.load_gather(gx2_v, [iv]))
            ai = jnp.max(plsc.load_gather(area_v, [iv]))
            bk = jnp.max(plsc.load_gather(keep_v, [iv]))

            @pl.when(bk > 0.5)
            def _():
                for j in range(13):
                    @pl.when(i < 16 * (j + 1))
                    def _(j=j):
                        sl = pl.ds(j * 16, 16)
                        yy1 = jnp.maximum(by1, gy1_v[sl])
                        xx1 = jnp.maximum(bx1, gx1_v[sl])
                        yy2 = jnp.minimum(by2, gy2_v[sl])
                        xx2 = jnp.minimum(bx2, gx2_v[sl])
                        inter = (jnp.maximum(yy2 - yy1, 0.0)
                                 * jnp.maximum(xx2 - xx1, 0.0))
                        iou = inter / (ai + area_v[sl] - inter + 1e-9)
                        colv = j * 16 + iota16
                        sup = (iou > _NMS_THR) & (colv > i)
                        keep_v[sl] = jnp.where(sup, 0.0, keep_v[sl])
            return x

        lax.fori_loop(0, _K, nms_i, jnp.int32(0))

        for j in range(_KPAD // 16):
            sl = pl.ds(j * 16, 16)
            if j < 13:
                fin = (keep_v[sl] > 0.5) & (gval_v[sl] > _SCORE_THR)
                ov_v[sl] = jnp.where(fin, gval_v[sl], 0.0)
                oy1_v[sl] = jnp.where(fin, gy1_v[sl], 0.0)
                ox1_v[sl] = jnp.where(fin, gx1_v[sl], 0.0)
                oy2_v[sl] = jnp.where(fin, gy2_v[sl], 0.0)
                ox2_v[sl] = jnp.where(fin, gx2_v[sl], 0.0)
            else:
                z = jnp.zeros((16,), jnp.float32)
                ov_v[sl] = z
                oy1_v[sl] = z
                ox1_v[sl] = z
                oy2_v[sl] = z
                ox2_v[sl] = z
        pltpu.sync_copy(ov_v, ov_hbm.at[c])
        pltpu.sync_copy(oy1_v, oy1_hbm.at[c])
        pltpu.sync_copy(ox1_v, ox1_hbm.at[c])
        pltpu.sync_copy(oy2_v, oy2_hbm.at[c])
        pltpu.sync_copy(ox2_v, ox2_hbm.at[c])


def _sc_stage(m, y1, x1, y2, x2):
    fo = jax.ShapeDtypeStruct((_C, _KPAD), jnp.float32)
    fN = lambda: pltpu.VMEM((_NPAD,), jnp.float32)
    fK = lambda: pltpu.VMEM((_KPAD,), jnp.float32)
    kern = pl.kernel(
        _sc_body,
        out_type=(fo, fo, fo, fo, fo),
        mesh=plsc.VectorSubcoreMesh(core_axis_name="c", subcore_axis_name="s",
                                    num_cores=2, num_subcores=16),
        compiler_params=pltpu.CompilerParams(use_tc_tiling_on_sc=False,
                                             needs_layout_passes=False),
        scratch_types=[
            fN(), fN(), fN(), fN(), fN(),
            pltpu.VMEM((_NPAD,), jnp.float32),
            pltpu.VMEM((_NPAD,), jnp.int32),
            pltpu.VMEM((_NPAD // 16,), jnp.float32),
            fK(), pltpu.VMEM((_KPAD,), jnp.int32),
            fK(), fK(), fK(), fK(),
            fK(), fK(),
            fK(), fK(), fK(), fK(), fK(),
        ],
    )
    return kern(m, y1, x1, y2, x2)


def kernel(proposed_roi_bboxes, predicted_roi_loc, predicted_roi_score):
    pad = _NPAD - _N
    scoreT = jnp.pad(predicted_roi_score.T, ((0, 0), (0, pad)))
    propT = jnp.pad(proposed_roi_bboxes.T, ((0, 0), (0, pad)))
    lr = predicted_roi_loc.reshape(_N, _C + 1, 4)[:, 1:, :]
    dyT = jnp.pad(lr[..., 0].T, ((0, 0), (0, pad)))
    dxT = jnp.pad(lr[..., 1].T, ((0, 0), (0, pad)))
    dhT = jnp.pad(lr[..., 2].T, ((0, 0), (0, pad)))
    dwT = jnp.pad(lr[..., 3].T, ((0, 0), (0, pad)))

    g = jax.ShapeDtypeStruct((_C, _NPAD), jnp.float32)
    m, y1, x1, y2, x2 = pl.pallas_call(
        _dense_body,
        out_shape=(g, g, g, g, g),
    )(scoreT, propT, dyT, dxT, dhT, dwT)

    ov, oy1, ox1, oy2, ox2 = _sc_stage(m, y1, x1, y2, x2)

    bboxes = jnp.stack([oy1, ox1, oy2, ox2], axis=-1)[:, :_K, :]
    bboxes = bboxes.reshape(_C * _K, 4)
    scores = ov[:, :_K].reshape(_C * _K)
    crow = jnp.arange(1, _C + 1, dtype=jnp.int32)[:, None]
    labels = jnp.where(ov[:, :_K] > 0.0, crow, 0).reshape(_C * _K)
    return bboxes, labels, scores
